# baseline (speedup 1.0000x reference)
import functools

import jax
import jax.numpy as jnp
from jax import lax
from jax.experimental import pallas as pl
from jax.experimental.pallas import tpu as pltpu
from jax.experimental.pallas import tpu_sc as plsc

_NUM_CLASSES = 1000
_D = 128
_N = 100000

_NC = 2
_NS = 16
_NW = _NC * _NS
_L = 16

_STEP = 128
_C = 3200
_NSTEPS = _C // _STEP
_NBUF = 5
_NGROUP = _NSTEPS // _NBUF
_LAST = _NW - 1
_TAIL = _N - _LAST * _C
_TFULL = _TAIL // _STEP
_TREM = _TAIL - _TFULL * _STEP
_NZ = 16


@functools.partial(
    pl.kernel,
    out_type=jax.ShapeDtypeStruct((_N, _D), jnp.float32),
    mesh=plsc.VectorSubcoreMesh(core_axis_name="c", subcore_axis_name="s"),
    scratch_types=[
        pltpu.VMEM((_C,), jnp.int32),
        pltpu.VMEM((_C,), jnp.int32),
        pltpu.VMEM((_C,), jnp.int32),
        pltpu.VMEM((_NBUF, _STEP, _D), jnp.float32),
        pltpu.VMEM_SHARED((_NUM_CLASSES + _NZ, _D), jnp.float32),
        pltpu.SemaphoreType.DMA((_NBUF,)),
        pltpu.SemaphoreType.DMA((_NBUF,)),
        pltpu.SemaphoreType.DMA,
    ],
)
def _masked_gather(y_hbm, m_hbm, tab_hbm, out_hbm, y_v, m_v, idx_v, rows_v,
                   tab_sp, sem_g, sem_s, sem_t):
    wid = lax.axis_index("s") * _NC + lax.axis_index("c")
    base = wid * _C

    @pl.when(lax.axis_index("s") == 0)
    def _():
        pltpu.async_copy(tab_hbm, tab_sp, sem_t)

    @pl.when(wid < _LAST)
    def _():
        pltpu.sync_copy(y_hbm.at[pl.ds(base, _C)], y_v)
        pltpu.sync_copy(m_hbm.at[pl.ds(base, _C)], m_v)

    @pl.when(wid == _LAST)
    def _():
        pltpu.sync_copy(y_hbm.at[pl.ds(base, _TAIL)], y_v.at[pl.ds(0, _TAIL)])
        pltpu.sync_copy(m_hbm.at[pl.ds(base, _TAIL)], m_v.at[pl.ds(0, _TAIL)])

    zrow = _NUM_CLASSES + lax.iota(jnp.int32, _L)

    def sel(i, carry):
        yv = y_v[pl.ds(i * _L, _L)]
        mv = m_v[pl.ds(i * _L, _L)]
        idx_v[pl.ds(i * _L, _L)] = jnp.where(mv > 0, yv, zrow)
        return carry

    lax.fori_loop(0, _C // _L, sel, 0)

    @pl.when(lax.axis_index("s") == 0)
    def _():
        pltpu.make_async_copy(tab_hbm, tab_sp, sem_t).wait()

    plsc.subcore_barrier()

    def start_gather(s, b):
        pltpu.async_copy(
            tab_sp.at[idx_v.at[pl.ds(s * _STEP, _STEP)]], rows_v.at[b],
            sem_g.at[b])

    def wait_gather(b):
        pltpu.make_async_copy(
            tab_sp.at[pl.ds(0, _STEP)], rows_v.at[b], sem_g.at[b]).wait()

    def start_scatter(s, b):
        pltpu.async_copy(
            rows_v.at[b], out_hbm.at[pl.ds(base + s * _STEP, _STEP)],
            sem_s.at[b])

    def wait_scatter(b):
        pltpu.make_async_copy(
            rows_v.at[b], out_hbm.at[pl.ds(base, _STEP)], sem_s.at[b]).wait()

    @pl.when(wid < _LAST)
    def _():
        for b in range(_NBUF - 2):
            start_gather(b, b)
        for s in range(2):
            start_gather(s + _NBUF - 2, s + _NBUF - 2)
            wait_gather(s)
            start_scatter(s, s)

        def round_(r, carry):
            for b in range(_NBUF):
                s = r * _NBUF + b + 2
                wait_scatter(b)
                start_gather(s + _NBUF - 2, b)
                wait_gather((b + 2) % _NBUF)
                start_scatter(s, (b + 2) % _NBUF)
            return carry

        lax.fori_loop(0, (_NSTEPS - _NBUF) // _NBUF, round_, 0)

        for s in range(_NSTEPS - 3, _NSTEPS):
            wait_gather(s % _NBUF)
            start_scatter(s, s % _NBUF)
        for b in range(_NBUF):
            wait_scatter(b)

    @pl.when(wid == _LAST)
    def _():
        for s in range(_TFULL):
            b = s % _NBUF
            start_gather(s, b)
            wait_gather(b)
            start_scatter(s, b)
            wait_scatter(b)
        pltpu.async_copy(
            tab_sp.at[idx_v.at[pl.ds(_TFULL * _STEP, _TREM)]],
            rows_v.at[0, pl.ds(0, _TREM)], sem_g.at[0])
        pltpu.make_async_copy(
            tab_sp.at[pl.ds(0, _TREM)], rows_v.at[0, pl.ds(0, _TREM)],
            sem_g.at[0]).wait()
        pltpu.async_copy(
            rows_v.at[0, pl.ds(0, _TREM)],
            out_hbm.at[pl.ds(base + _TFULL * _STEP, _TREM)], sem_s.at[0])
        pltpu.make_async_copy(
            rows_v.at[0, pl.ds(0, _TREM)],
            out_hbm.at[pl.ds(base, _TREM)], sem_s.at[0]).wait()


def kernel(y, mask, emb_weight):
    tab = jnp.pad(emb_weight, ((0, _NZ), (0, 0)))
    return _masked_gather(y, mask.astype(jnp.int32), tab)


# speedup vs baseline: 1.0113x; 1.0113x over previous
import functools

import jax
import jax.numpy as jnp
from jax import lax
from jax.experimental import pallas as pl
from jax.experimental.pallas import tpu as pltpu
from jax.experimental.pallas import tpu_sc as plsc

_NUM_CLASSES = 1000
_D = 128
_N = 100000

_NC = 2
_NS = 16
_NW = _NC * _NS
_L = 16

_STEP = 128
_C = 3200
_NSTEPS = _C // _STEP
_NBUF = 5
_NGROUP = _NSTEPS // _NBUF
_LAST = _NW - 1
_TAIL = _N - _LAST * _C
_TFULL = _TAIL // _STEP
_TREM = _TAIL - _TFULL * _STEP
_NZ = 16


@functools.partial(
    pl.kernel,
    out_type=jax.ShapeDtypeStruct((_N, _D), jnp.float32),
    mesh=plsc.VectorSubcoreMesh(core_axis_name="c", subcore_axis_name="s"),
    scratch_types=[
        pltpu.VMEM((_C,), jnp.int32),
        pltpu.VMEM((_C,), jnp.int32),
        pltpu.VMEM((_C,), jnp.int32),
        pltpu.VMEM((_NBUF, _STEP, _D), jnp.float32),
        pltpu.VMEM_SHARED((_NUM_CLASSES + _NZ, _D), jnp.float32),
        pltpu.SemaphoreType.DMA((_NBUF,)),
        pltpu.SemaphoreType.DMA((_NBUF,)),
        pltpu.SemaphoreType.DMA,
    ],
)
def _masked_gather(y_hbm, m_hbm, tab_hbm, out_hbm, y_v, m_v, idx_v, rows_v,
                   tab_sp, sem_g, sem_s, sem_t):
    wid = lax.axis_index("s") * _NC + lax.axis_index("c")
    base = wid * _C

    @pl.when(lax.axis_index("s") == 0)
    def _():
        pltpu.async_copy(tab_hbm, tab_sp.at[pl.ds(0, _NUM_CLASSES)], sem_t)

    @pl.when(wid < _LAST)
    def _():
        pltpu.sync_copy(y_hbm.at[pl.ds(base, _C)], y_v)
        pltpu.sync_copy(m_hbm.at[pl.ds(base, _C)], m_v)

    @pl.when(wid == _LAST)
    def _():
        pltpu.sync_copy(y_hbm.at[pl.ds(base, _TAIL)], y_v.at[pl.ds(0, _TAIL)])
        pltpu.sync_copy(m_hbm.at[pl.ds(base, _TAIL)], m_v.at[pl.ds(0, _TAIL)])

    zrow = _NUM_CLASSES + lax.iota(jnp.int32, _L)

    def sel(i, carry):
        yv = y_v[pl.ds(i * _L, _L)]
        mv = m_v[pl.ds(i * _L, _L)]
        idx_v[pl.ds(i * _L, _L)] = jnp.where(mv != 0, yv, zrow)
        return carry

    lax.fori_loop(0, _C // _L, sel, 0)

    @pl.when(lax.axis_index("s") == 0)
    def _():
        for r in range(_NZ):
            for j in range(_D // _L):
                rows_v[0, r, pl.ds(j * _L, _L)] = jnp.zeros((_L,), jnp.float32)
        pltpu.sync_copy(rows_v.at[0, pl.ds(0, _NZ)],
                        tab_sp.at[pl.ds(_NUM_CLASSES, _NZ)])
        pltpu.make_async_copy(tab_hbm, tab_sp.at[pl.ds(0, _NUM_CLASSES)],
                              sem_t).wait()

    plsc.subcore_barrier()

    def start_gather(s, b):
        pltpu.async_copy(
            tab_sp.at[idx_v.at[pl.ds(s * _STEP, _STEP)]], rows_v.at[b],
            sem_g.at[b])

    def wait_gather(b):
        pltpu.make_async_copy(
            tab_sp.at[pl.ds(0, _STEP)], rows_v.at[b], sem_g.at[b]).wait()

    def start_scatter(s, b):
        pltpu.async_copy(
            rows_v.at[b], out_hbm.at[pl.ds(base + s * _STEP, _STEP)],
            sem_s.at[b])

    def wait_scatter(b):
        pltpu.make_async_copy(
            rows_v.at[b], out_hbm.at[pl.ds(base, _STEP)], sem_s.at[b]).wait()

    @pl.when(wid < _LAST)
    def _():
        for b in range(_NBUF - 2):
            start_gather(b, b)
        for s in range(2):
            start_gather(s + _NBUF - 2, s + _NBUF - 2)
            wait_gather(s)
            start_scatter(s, s)

        def round_(r, carry):
            for b in range(_NBUF):
                s = r * _NBUF + b + 2
                wait_scatter(b)
                start_gather(s + _NBUF - 2, b)
                wait_gather((b + 2) % _NBUF)
                start_scatter(s, (b + 2) % _NBUF)
            return carry

        lax.fori_loop(0, (_NSTEPS - _NBUF) // _NBUF, round_, 0)

        for s in range(_NSTEPS - 3, _NSTEPS):
            wait_gather(s % _NBUF)
            start_scatter(s, s % _NBUF)
        for b in range(_NBUF):
            wait_scatter(b)

    @pl.when(wid == _LAST)
    def _():
        for s in range(_TFULL):
            b = s % _NBUF
            start_gather(s, b)
            wait_gather(b)
            start_scatter(s, b)
            wait_scatter(b)
        pltpu.async_copy(
            tab_sp.at[idx_v.at[pl.ds(_TFULL * _STEP, _TREM)]],
            rows_v.at[0, pl.ds(0, _TREM)], sem_g.at[0])
        pltpu.make_async_copy(
            tab_sp.at[pl.ds(0, _TREM)], rows_v.at[0, pl.ds(0, _TREM)],
            sem_g.at[0]).wait()
        pltpu.async_copy(
            rows_v.at[0, pl.ds(0, _TREM)],
            out_hbm.at[pl.ds(base + _TFULL * _STEP, _TREM)], sem_s.at[0])
        pltpu.make_async_copy(
            rows_v.at[0, pl.ds(0, _TREM)],
            out_hbm.at[pl.ds(base, _TREM)], sem_s.at[0]).wait()


def kernel(y, mask, emb_weight):
    return _masked_gather(y, mask.astype(jnp.int32), emb_weight)


# speedup vs baseline: 1.0263x; 1.0148x over previous
import functools

import jax
import jax.numpy as jnp
from jax import lax
from jax.experimental import pallas as pl
from jax.experimental.pallas import tpu as pltpu
from jax.experimental.pallas import tpu_sc as plsc

_NUM_CLASSES = 1000
_D = 128
_N = 100000

_NC = 2
_NS = 16
_NW = _NC * _NS
_L = 16

_STEP = 128
_C = 3200
_NSTEPS = _C // _STEP
_NBUF = 5
_NGROUP = _NSTEPS // _NBUF
_LAST = _NW - 1
_TAIL = _N - _LAST * _C
_TFULL = _TAIL // _STEP
_TREM = _TAIL - _TFULL * _STEP
_NZ = 16


@functools.partial(
    pl.kernel,
    out_type=jax.ShapeDtypeStruct((_N, _D), jnp.float32),
    mesh=plsc.VectorSubcoreMesh(core_axis_name="c", subcore_axis_name="s"),
    scratch_types=[
        pltpu.VMEM((_C,), jnp.int32),
        pltpu.VMEM((_C,), jnp.int32),
        pltpu.VMEM((_C,), jnp.int32),
        pltpu.VMEM((_NBUF, _STEP, _D), jnp.float32),
        pltpu.VMEM_SHARED((_NUM_CLASSES + _NZ, _D), jnp.float32),
        pltpu.SemaphoreType.DMA((_NBUF,)),
        pltpu.SemaphoreType.DMA((_NBUF,)),
        pltpu.SemaphoreType.DMA,
    ],
)
def _masked_gather(y_hbm, m_hbm, tab_hbm, out_hbm, y_v, m_v, idx_v, rows_v,
                   tab_sp, sem_g, sem_s, sem_t):
    wid = lax.axis_index("s") * _NC + lax.axis_index("c")
    base = wid * _C

    @pl.when(lax.axis_index("s") == 0)
    def _():
        pltpu.async_copy(tab_hbm, tab_sp.at[pl.ds(0, _NUM_CLASSES)], sem_t)

    @pl.when(wid < _LAST)
    def _():
        pltpu.sync_copy(y_hbm.at[pl.ds(base, _C)], y_v)
        pltpu.sync_copy(m_hbm.at[pl.ds(base, _C)], m_v)

    @pl.when(wid == _LAST)
    def _():
        pltpu.sync_copy(y_hbm.at[pl.ds(base, _TAIL)], y_v.at[pl.ds(0, _TAIL)])
        pltpu.sync_copy(m_hbm.at[pl.ds(base, _TAIL)], m_v.at[pl.ds(0, _TAIL)])

    zrow = _NUM_CLASSES + lax.iota(jnp.int32, _L)

    def sel(i, carry):
        yv = y_v[pl.ds(i * _L, _L)]
        mv = m_v[pl.ds(i * _L, _L)]
        idx_v[pl.ds(i * _L, _L)] = jnp.where(mv != 0, yv, zrow)
        return carry

    _PRE = _NBUF * _STEP // _L
    npre = jnp.where(wid < _LAST, _PRE, _TAIL // _L)
    lax.fori_loop(0, npre, sel, 0)

    @pl.when(lax.axis_index("s") == 0)
    def _():
        for r in range(_NZ):
            for j in range(_D // _L):
                rows_v[0, r, pl.ds(j * _L, _L)] = jnp.zeros((_L,), jnp.float32)
        pltpu.sync_copy(rows_v.at[0, pl.ds(0, _NZ)],
                        tab_sp.at[pl.ds(_NUM_CLASSES, _NZ)])
        pltpu.make_async_copy(tab_hbm, tab_sp.at[pl.ds(0, _NUM_CLASSES)],
                              sem_t).wait()

    plsc.subcore_barrier()

    def start_gather(s, b):
        pltpu.async_copy(
            tab_sp.at[idx_v.at[pl.ds(s * _STEP, _STEP)]], rows_v.at[b],
            sem_g.at[b])

    def wait_gather(b):
        pltpu.make_async_copy(
            tab_sp.at[pl.ds(0, _STEP)], rows_v.at[b], sem_g.at[b]).wait()

    def start_scatter(s, b):
        pltpu.async_copy(
            rows_v.at[b], out_hbm.at[pl.ds(base + s * _STEP, _STEP)],
            sem_s.at[b])

    def wait_scatter(b):
        pltpu.make_async_copy(
            rows_v.at[b], out_hbm.at[pl.ds(base, _STEP)], sem_s.at[b]).wait()

    @pl.when(wid < _LAST)
    def _():
        for b in range(_NBUF):
            start_gather(b, b)
        lax.fori_loop(_PRE, _C // _L, sel, 0)
        for s in range(2):
            wait_gather(s)
            start_scatter(s, s)

        def round_(r, carry):
            for b in range(_NBUF):
                s = r * _NBUF + b + 2
                wait_scatter(b)
                start_gather(s + _NBUF - 2, b)
                wait_gather((b + 2) % _NBUF)
                start_scatter(s, (b + 2) % _NBUF)
            return carry

        lax.fori_loop(0, (_NSTEPS - _NBUF) // _NBUF, round_, 0)

        for s in range(_NSTEPS - 3, _NSTEPS):
            wait_gather(s % _NBUF)
            start_scatter(s, s % _NBUF)
        for b in range(_NBUF):
            wait_scatter(b)

    @pl.when(wid == _LAST)
    def _():
        for s in range(_TFULL):
            b = s % _NBUF
            start_gather(s, b)
            wait_gather(b)
            start_scatter(s, b)
            wait_scatter(b)
        pltpu.async_copy(
            tab_sp.at[idx_v.at[pl.ds(_TFULL * _STEP, _TREM)]],
            rows_v.at[0, pl.ds(0, _TREM)], sem_g.at[0])
        pltpu.make_async_copy(
            tab_sp.at[pl.ds(0, _TREM)], rows_v.at[0, pl.ds(0, _TREM)],
            sem_g.at[0]).wait()
        pltpu.async_copy(
            rows_v.at[0, pl.ds(0, _TREM)],
            out_hbm.at[pl.ds(base + _TFULL * _STEP, _TREM)], sem_s.at[0])
        pltpu.make_async_copy(
            rows_v.at[0, pl.ds(0, _TREM)],
            out_hbm.at[pl.ds(base, _TREM)], sem_s.at[0]).wait()


def kernel(y, mask, emb_weight):
    return _masked_gather(y, mask.astype(jnp.int32), emb_weight)
